# Initial kernel scaffold; baseline (speedup 1.0000x reference)
import jax
import jax.numpy as jnp
from jax.experimental import pallas as pl


def kernel(hidden_states, Wr, W1, b1, W2, b2, A, B):
    raise NotImplementedError("write your pallas kernel here")


# speedup vs baseline: 2.0714x; 2.0714x over previous
import jax
import jax.numpy as jnp
from jax.experimental import pallas as pl
from jax.experimental.pallas import tpu as pltpu

D_MODEL = 1024
D_FF = 4096
N_EXP = 5
N_LORA = 4
RANK = 16
LORA_SCALE = 2.0
AUX_W = 0.01

TB = 256
FB = 2048
NE_PAD = 8


def _moe_kernel(x_ref, wr_ref, w1_ref, b1_ref, w2t_ref, b2_ref, acat_ref,
                bstk_ref, out_ref, stats_ref, acc_ref):
    i = pl.program_id(0)
    j = pl.program_id(1)
    nj = pl.num_programs(1)
    x = x_ref[...]

    h = jnp.dot(x, w1_ref[...].T, preferred_element_type=jnp.float32)
    h = h + b1_ref[...]
    h = h * 0.5 * (1.0 + jax.lax.erf(h * 0.7071067811865476))
    partial = jnp.dot(h, w2t_ref[...], preferred_element_type=jnp.float32)

    @pl.when(j == 0)
    def _():
        logits = jnp.dot(x, wr_ref[...].T, preferred_element_type=jnp.float32)
        col8 = jax.lax.broadcasted_iota(jnp.int32, (TB, NE_PAD), 1)
        logits = jnp.where(col8 < N_EXP, logits, -jnp.inf)
        m = jnp.max(logits, axis=-1, keepdims=True)
        e = jnp.exp(logits - m)
        probs = e / jnp.sum(e, axis=-1, keepdims=True)
        sel = jnp.argmax(logits, axis=1).reshape(TB, 1)
        onehot = (col8 == sel).astype(jnp.float32)

        pc = jnp.concatenate(
            [jnp.sum(probs, axis=0, keepdims=True),
             jnp.sum(onehot, axis=0, keepdims=True)], axis=0)

        @pl.when(i == 0)
        def _():
            stats_ref[...] = pc

        @pl.when(i > 0)
        def _():
            stats_ref[...] = stats_ref[...] + pc

        t = jnp.dot(x, acat_ref[...].T, preferred_element_type=jnp.float32)
        colL = jax.lax.broadcasted_iota(jnp.int32, (TB, N_LORA * RANK), 1)
        eid = colL // RANK + 1
        tm = jnp.where(sel == eid, t, 0.0)
        lora = jnp.dot(tm, bstk_ref[...],
                       preferred_element_type=jnp.float32) * LORA_SCALE
        acc_ref[...] = partial + lora + b2_ref[...]

    @pl.when(j > 0)
    def _():
        acc_ref[...] = acc_ref[...] + partial

    @pl.when(j == nj - 1)
    def _():
        out_ref[...] = acc_ref[...]


def kernel(hidden_states, Wr, W1, b1, W2, b2, A, B):
    Bsz, S, D = hidden_states.shape
    T = Bsz * S
    x = hidden_states.reshape(T, D)
    wr_pad = jnp.zeros((NE_PAD, D), Wr.dtype).at[:N_EXP].set(Wr)
    w2t = W2.T
    b1r = b1.reshape(1, D_FF)
    b2r = b2.reshape(1, D)
    acat = A.reshape(N_LORA * RANK, D)
    bstk = jnp.transpose(B, (0, 2, 1)).reshape(N_LORA * RANK, D)

    grid = (T // TB, D_FF // FB)
    out, stats = pl.pallas_call(
        _moe_kernel,
        grid=grid,
        in_specs=[
            pl.BlockSpec((TB, D), lambda i, j: (i, 0)),
            pl.BlockSpec((NE_PAD, D), lambda i, j: (0, 0)),
            pl.BlockSpec((FB, D), lambda i, j: (j, 0)),
            pl.BlockSpec((1, FB), lambda i, j: (0, j)),
            pl.BlockSpec((FB, D), lambda i, j: (j, 0)),
            pl.BlockSpec((1, D), lambda i, j: (0, 0)),
            pl.BlockSpec((N_LORA * RANK, D), lambda i, j: (0, 0)),
            pl.BlockSpec((N_LORA * RANK, D), lambda i, j: (0, 0)),
        ],
        out_specs=[
            pl.BlockSpec((TB, D), lambda i, j: (i, 0)),
            pl.BlockSpec((2, NE_PAD), lambda i, j: (0, 0)),
        ],
        out_shape=[
            jax.ShapeDtypeStruct((T, D), jnp.float32),
            jax.ShapeDtypeStruct((2, NE_PAD), jnp.float32),
        ],
        scratch_shapes=[pltpu.VMEM((TB, D), jnp.float32)],
        compiler_params=pltpu.CompilerParams(
            dimension_semantics=("arbitrary", "arbitrary")),
    )(x, wr_pad, W1, b1r, w2t, b2r, acat, bstk)

    probs_mean = stats[0, :N_EXP] / T
    counts_mean = stats[1, :N_EXP] / T
    aux = jnp.sum(probs_mean * counts_mean) * N_EXP * AUX_W
    return out.reshape(Bsz, S, D), aux


# speedup vs baseline: 2.8500x; 1.3758x over previous
import jax
import jax.numpy as jnp
from jax.experimental import pallas as pl
from jax.experimental.pallas import tpu as pltpu

D_MODEL = 1024
D_FF = 4096
N_EXP = 5
N_LORA = 4
RANK = 16
LORA_SCALE = 2.0
AUX_W = 0.01

TB = 256
NE_PAD = 8


def _moe_kernel(x_ref, wr_ref, w1_ref, b1_ref, w2t_ref, b2_ref, acat_ref,
                bstk_ref, out_ref, stats_ref):
    i = pl.program_id(0)
    x = x_ref[...]

    h = jnp.dot(x, w1_ref[...].T, preferred_element_type=jnp.float32)
    h = h + b1_ref[...]
    h = h * 0.5 * (1.0 + jax.lax.erf(h * 0.7071067811865476))
    base = jnp.dot(h, w2t_ref[...], preferred_element_type=jnp.float32)

    logits = jnp.dot(x, wr_ref[...].T, preferred_element_type=jnp.float32)
    col8 = jax.lax.broadcasted_iota(jnp.int32, (TB, NE_PAD), 1)
    logits = jnp.where(col8 < N_EXP, logits, -jnp.inf)
    m = jnp.max(logits, axis=-1, keepdims=True)
    e = jnp.exp(logits - m)
    probs = e / jnp.sum(e, axis=-1, keepdims=True)
    sel = jnp.argmax(logits, axis=1).reshape(TB, 1)
    onehot = (col8 == sel).astype(jnp.float32)

    pc = jnp.concatenate(
        [jnp.sum(probs, axis=0, keepdims=True),
         jnp.sum(onehot, axis=0, keepdims=True)], axis=0)

    @pl.when(i == 0)
    def _():
        stats_ref[...] = pc

    @pl.when(i > 0)
    def _():
        stats_ref[...] = stats_ref[...] + pc

    t = jnp.dot(x, acat_ref[...].T, preferred_element_type=jnp.float32)
    colL = jax.lax.broadcasted_iota(jnp.int32, (TB, N_LORA * RANK), 1)
    eid = colL // RANK + 1
    tm = jnp.where(sel == eid, t, 0.0)
    lora = jnp.dot(tm, bstk_ref[...],
                   preferred_element_type=jnp.float32) * LORA_SCALE
    out_ref[...] = base + lora + b2_ref[...]


def kernel(hidden_states, Wr, W1, b1, W2, b2, A, B):
    Bsz, S, D = hidden_states.shape
    T = Bsz * S
    x = hidden_states.reshape(T, D)
    wr_pad = jnp.zeros((NE_PAD, D), Wr.dtype).at[:N_EXP].set(Wr)
    w2t = W2.T
    b1r = b1.reshape(1, D_FF)
    b2r = b2.reshape(1, D)
    acat = A.reshape(N_LORA * RANK, D)
    bstk = jnp.transpose(B, (0, 2, 1)).reshape(N_LORA * RANK, D)

    grid = (T // TB,)
    out, stats = pl.pallas_call(
        _moe_kernel,
        grid=grid,
        in_specs=[
            pl.BlockSpec((TB, D), lambda i: (i, 0)),
            pl.BlockSpec((NE_PAD, D), lambda i: (0, 0)),
            pl.BlockSpec((D_FF, D), lambda i: (0, 0)),
            pl.BlockSpec((1, D_FF), lambda i: (0, 0)),
            pl.BlockSpec((D_FF, D), lambda i: (0, 0)),
            pl.BlockSpec((1, D), lambda i: (0, 0)),
            pl.BlockSpec((N_LORA * RANK, D), lambda i: (0, 0)),
            pl.BlockSpec((N_LORA * RANK, D), lambda i: (0, 0)),
        ],
        out_specs=[
            pl.BlockSpec((TB, D), lambda i: (i, 0)),
            pl.BlockSpec((2, NE_PAD), lambda i: (0, 0)),
        ],
        out_shape=[
            jax.ShapeDtypeStruct((T, D), jnp.float32),
            jax.ShapeDtypeStruct((2, NE_PAD), jnp.float32),
        ],
        compiler_params=pltpu.CompilerParams(
            dimension_semantics=("arbitrary",)),
    )(x, wr_pad, W1, b1r, w2t, b2r, acat, bstk)

    probs_mean = stats[0, :N_EXP] / T
    counts_mean = stats[1, :N_EXP] / T
    aux = jnp.sum(probs_mean * counts_mean) * N_EXP * AUX_W
    return out.reshape(Bsz, S, D), aux


# speedup vs baseline: 3.7400x; 1.3123x over previous
import jax
import jax.numpy as jnp
from jax.experimental import pallas as pl
from jax.experimental.pallas import tpu as pltpu

D_MODEL = 1024
D_FF = 4096
N_EXP = 5
N_LORA = 4
RANK = 16
LORA_SCALE = 2.0
AUX_W = 0.01

TB = 512
NE_PAD = 8


def _moe_kernel(x_ref, wr_ref, w1_ref, b1_ref, w2b_ref, b2_ref, acat_ref,
                bstk_ref, out_ref, stats_ref):
    i = pl.program_id(0)
    x = x_ref[...]

    xb = x.astype(jnp.bfloat16)
    h = jnp.dot(xb, w1_ref[...].T, preferred_element_type=jnp.float32)
    h = h + b1_ref[...]
    h = h * 0.5 * (1.0 + jax.lax.erf(h * 0.7071067811865476))
    base = jax.lax.dot_general(
        h.astype(jnp.bfloat16), w2b_ref[...],
        dimension_numbers=(((1,), (1,)), ((), ())),
        preferred_element_type=jnp.float32)

    logits = jnp.dot(x, wr_ref[...].T, preferred_element_type=jnp.float32)
    col8 = jax.lax.broadcasted_iota(jnp.int32, (TB, NE_PAD), 1)
    logits = jnp.where(col8 < N_EXP, logits, -jnp.inf)
    m = jnp.max(logits, axis=-1, keepdims=True)
    e = jnp.exp(logits - m)
    probs = e / jnp.sum(e, axis=-1, keepdims=True)
    sel = jnp.argmax(logits, axis=1).reshape(TB, 1)
    onehot = (col8 == sel).astype(jnp.float32)

    pc = jnp.concatenate(
        [jnp.sum(probs, axis=0, keepdims=True),
         jnp.sum(onehot, axis=0, keepdims=True)], axis=0)

    @pl.when(i == 0)
    def _():
        stats_ref[...] = pc

    @pl.when(i > 0)
    def _():
        stats_ref[...] = stats_ref[...] + pc

    t = jnp.dot(xb, acat_ref[...].T, preferred_element_type=jnp.float32)
    colL = jax.lax.broadcasted_iota(jnp.int32, (TB, N_LORA * RANK), 1)
    eid = colL // RANK + 1
    tm = jnp.where(sel == eid, t, 0.0)
    lora = jnp.dot(tm, bstk_ref[...],
                   preferred_element_type=jnp.float32) * LORA_SCALE
    out_ref[...] = base + lora + b2_ref[...]


def kernel(hidden_states, Wr, W1, b1, W2, b2, A, B):
    Bsz, S, D = hidden_states.shape
    T = Bsz * S
    x = hidden_states.reshape(T, D)
    wr_pad = jnp.zeros((NE_PAD, D), Wr.dtype).at[:N_EXP].set(Wr)
    w1b = W1.astype(jnp.bfloat16)
    w2b = W2.astype(jnp.bfloat16)
    b1r = b1.reshape(1, D_FF)
    b2r = b2.reshape(1, D)
    acat = A.reshape(N_LORA * RANK, D).astype(jnp.bfloat16)
    bstk = jnp.transpose(B, (0, 2, 1)).reshape(N_LORA * RANK, D).astype(jnp.bfloat16)

    grid = (T // TB,)
    out, stats = pl.pallas_call(
        _moe_kernel,
        grid=grid,
        in_specs=[
            pl.BlockSpec((TB, D), lambda i: (i, 0)),
            pl.BlockSpec((NE_PAD, D), lambda i: (0, 0)),
            pl.BlockSpec((D_FF, D), lambda i: (0, 0)),
            pl.BlockSpec((1, D_FF), lambda i: (0, 0)),
            pl.BlockSpec((D, D_FF), lambda i: (0, 0)),
            pl.BlockSpec((1, D), lambda i: (0, 0)),
            pl.BlockSpec((N_LORA * RANK, D), lambda i: (0, 0)),
            pl.BlockSpec((N_LORA * RANK, D), lambda i: (0, 0)),
        ],
        out_specs=[
            pl.BlockSpec((TB, D), lambda i: (i, 0)),
            pl.BlockSpec((2, NE_PAD), lambda i: (0, 0)),
        ],
        out_shape=[
            jax.ShapeDtypeStruct((T, D), jnp.float32),
            jax.ShapeDtypeStruct((2, NE_PAD), jnp.float32),
        ],
        compiler_params=pltpu.CompilerParams(
            dimension_semantics=("arbitrary",)),
    )(x, wr_pad, w1b, b1r, w2b, b2r, acat, bstk)

    probs_mean = stats[0, :N_EXP] / T
    counts_mean = stats[1, :N_EXP] / T
    aux = jnp.sum(probs_mean * counts_mean) * N_EXP * AUX_W
    return out.reshape(Bsz, S, D), aux


# speedup vs baseline: 3.8831x; 1.0383x over previous
import jax
import jax.numpy as jnp
from jax.experimental import pallas as pl
from jax.experimental.pallas import tpu as pltpu

D_MODEL = 1024
D_FF = 4096
N_EXP = 5
N_LORA = 4
RANK = 16
LORA_SCALE = 2.0
AUX_W = 0.01

TB = 1024
NE_PAD = 8


def _moe_kernel(x_ref, wr_ref, w1_ref, b1_ref, w2b_ref, b2_ref, acat_ref,
                bstk_ref, out_ref, stats_ref):
    i = pl.program_id(0)
    x = x_ref[...]

    xb = x.astype(jnp.bfloat16)
    h = jnp.dot(xb, w1_ref[...].T, preferred_element_type=jnp.float32)
    h = h + b1_ref[...]
    h = h * 0.5 * (1.0 + jax.lax.erf(h * 0.7071067811865476))
    base = jax.lax.dot_general(
        h.astype(jnp.bfloat16), w2b_ref[...],
        dimension_numbers=(((1,), (1,)), ((), ())),
        preferred_element_type=jnp.float32)

    logits = jnp.dot(x, wr_ref[...].T, preferred_element_type=jnp.float32)
    col8 = jax.lax.broadcasted_iota(jnp.int32, (TB, NE_PAD), 1)
    logits = jnp.where(col8 < N_EXP, logits, -jnp.inf)
    m = jnp.max(logits, axis=-1, keepdims=True)
    e = jnp.exp(logits - m)
    probs = e / jnp.sum(e, axis=-1, keepdims=True)
    sel = jnp.argmax(logits, axis=1).reshape(TB, 1)
    onehot = (col8 == sel).astype(jnp.float32)

    pc = jnp.concatenate(
        [jnp.sum(probs, axis=0, keepdims=True),
         jnp.sum(onehot, axis=0, keepdims=True)], axis=0)

    @pl.when(i == 0)
    def _():
        stats_ref[...] = pc

    @pl.when(i > 0)
    def _():
        stats_ref[...] = stats_ref[...] + pc

    t = jnp.dot(xb, acat_ref[...].T, preferred_element_type=jnp.float32)
    colL = jax.lax.broadcasted_iota(jnp.int32, (TB, N_LORA * RANK), 1)
    eid = colL // RANK + 1
    tm = jnp.where(sel == eid, t, 0.0)
    lora = jnp.dot(tm, bstk_ref[...],
                   preferred_element_type=jnp.float32) * LORA_SCALE
    out_ref[...] = base + lora + b2_ref[...]


def kernel(hidden_states, Wr, W1, b1, W2, b2, A, B):
    Bsz, S, D = hidden_states.shape
    T = Bsz * S
    x = hidden_states.reshape(T, D)
    wr_pad = jnp.zeros((NE_PAD, D), Wr.dtype).at[:N_EXP].set(Wr)
    w1b = W1.astype(jnp.bfloat16)
    w2b = W2.astype(jnp.bfloat16)
    b1r = b1.reshape(1, D_FF)
    b2r = b2.reshape(1, D)
    acat = A.reshape(N_LORA * RANK, D).astype(jnp.bfloat16)
    bstk = jnp.transpose(B, (0, 2, 1)).reshape(N_LORA * RANK, D).astype(jnp.bfloat16)

    grid = (T // TB,)
    out, stats = pl.pallas_call(
        _moe_kernel,
        grid=grid,
        in_specs=[
            pl.BlockSpec((TB, D), lambda i: (i, 0)),
            pl.BlockSpec((NE_PAD, D), lambda i: (0, 0)),
            pl.BlockSpec((D_FF, D), lambda i: (0, 0)),
            pl.BlockSpec((1, D_FF), lambda i: (0, 0)),
            pl.BlockSpec((D, D_FF), lambda i: (0, 0)),
            pl.BlockSpec((1, D), lambda i: (0, 0)),
            pl.BlockSpec((N_LORA * RANK, D), lambda i: (0, 0)),
            pl.BlockSpec((N_LORA * RANK, D), lambda i: (0, 0)),
        ],
        out_specs=[
            pl.BlockSpec((TB, D), lambda i: (i, 0)),
            pl.BlockSpec((2, NE_PAD), lambda i: (0, 0)),
        ],
        out_shape=[
            jax.ShapeDtypeStruct((T, D), jnp.float32),
            jax.ShapeDtypeStruct((2, NE_PAD), jnp.float32),
        ],
        compiler_params=pltpu.CompilerParams(
            dimension_semantics=("arbitrary",)),
    )(x, wr_pad, w1b, b1r, w2b, b2r, acat, bstk)

    probs_mean = stats[0, :N_EXP] / T
    counts_mean = stats[1, :N_EXP] / T
    aux = jnp.sum(probs_mean * counts_mean) * N_EXP * AUX_W
    return out.reshape(Bsz, S, D), aux
